# baseline (speedup 1.0000x reference)
import functools

import jax
import jax.numpy as jnp
from jax import lax
from jax.experimental import pallas as pl
from jax.experimental.pallas import tpu as pltpu
from jax.experimental.pallas import tpu_sc as plsc

_EMBED_DIM = 64
_NUM_CORES = 2
_NUM_SUBCORES = 16
_NW = _NUM_CORES * _NUM_SUBCORES

_G = 128


def _make_gather_kernel(n_total: int, d: int):
    b_per_w = n_total // _NW
    n_chunks = b_per_w // _G
    mesh = plsc.VectorSubcoreMesh(core_axis_name="c", subcore_axis_name="s")

    @functools.partial(
        pl.kernel,
        mesh=mesh,
        out_type=jax.ShapeDtypeStruct((n_total, d), jnp.float32),
        scratch_types=[
            pltpu.VMEM((n_chunks, _G), jnp.int32),
            pltpu.VMEM((_G, d), jnp.float32),
            pltpu.SemaphoreType.DMA,
        ],
        compiler_params=pltpu.CompilerParams(use_tc_tiling_on_sc=False),
    )
    def k(idx_hbm, table_hbm, out_hbm, idx_v, rows_v, sem):
        wid = lax.axis_index("s") * _NUM_CORES + lax.axis_index("c")
        base = wid * b_per_w
        pltpu.sync_copy(idx_hbm.at[wid], idx_v)

        def body(j, carry):
            pltpu.async_copy(table_hbm.at[idx_v.at[j]], rows_v, sem).wait()
            pltpu.sync_copy(rows_v, out_hbm.at[pl.ds(base + j * _G, _G)])
            return carry

        lax.fori_loop(0, n_chunks, body, 0)

    return k


def kernel(x, weight):
    batch, seq = x.shape
    n_total = batch * seq
    d = weight.shape[1]
    idx = x.reshape(_NW, n_total // (_NW * _G), _G).astype(jnp.int32)
    out = _make_gather_kernel(n_total, d)(idx, weight)
    return out.reshape(batch, seq, d)


# speedup vs baseline: 1.1171x; 1.1171x over previous
import functools

import jax
import jax.numpy as jnp
from jax import lax
from jax.experimental import pallas as pl
from jax.experimental.pallas import tpu as pltpu
from jax.experimental.pallas import tpu_sc as plsc

_NUM_CORES = 2
_NUM_SUBCORES = 16
_NW = _NUM_CORES * _NUM_SUBCORES

_G = 128
_K = 4


def _make_gather_kernel(n_total: int, d: int):
    b_per_w = n_total // _NW
    n_chunks = b_per_w // _G
    n_groups = n_chunks // _K
    assert n_chunks % _K == 0 and n_groups % 2 == 0 and n_groups >= 4
    mesh = plsc.VectorSubcoreMesh(core_axis_name="c", subcore_axis_name="s")

    @functools.partial(
        pl.kernel,
        mesh=mesh,
        out_type=jax.ShapeDtypeStruct((n_total, d), jnp.float32),
        scratch_types=[
            pltpu.VMEM((n_chunks, _G), jnp.int32),
            pltpu.VMEM((2 * _K, _G, d), jnp.float32),
            pltpu.SemaphoreType.DMA((2 * _K,)),
            pltpu.SemaphoreType.DMA((2 * _K,)),
        ],
        compiler_params=pltpu.CompilerParams(use_tc_tiling_on_sc=False),
    )
    def k(idx_hbm, table_hbm, out_hbm, idx_v, rows_v, gsem, osem):
        wid = lax.axis_index("s") * _NUM_CORES + lax.axis_index("c")
        base = wid * b_per_w
        pltpu.sync_copy(idx_hbm.at[wid], idx_v)

        def fire_gather(i, s):
            for b in range(_K):
                pltpu.async_copy(
                    table_hbm.at[idx_v.at[i * _K + b]],
                    rows_v.at[s * _K + b],
                    gsem.at[s * _K + b],
                )

        def drain_gather(i, s):
            for b in range(_K):
                pltpu.make_async_copy(
                    table_hbm.at[idx_v.at[i * _K + b]],
                    rows_v.at[s * _K + b],
                    gsem.at[s * _K + b],
                ).wait()

        def fire_out(i, s):
            for b in range(_K):
                pltpu.async_copy(
                    rows_v.at[s * _K + b],
                    out_hbm.at[pl.ds(base + (i * _K + b) * _G, _G)],
                    osem.at[s * _K + b],
                )

        def drain_out(i, s):
            for b in range(_K):
                pltpu.make_async_copy(
                    rows_v.at[s * _K + b],
                    out_hbm.at[pl.ds(base + (i * _K + b) * _G, _G)],
                    osem.at[s * _K + b],
                ).wait()

        fire_gather(0, 0)
        fire_gather(1, 1)
        drain_gather(0, 0)
        fire_out(0, 0)

        def steady(i2, carry):
            for g in range(2):
                i = 2 * i2 + 1 + g
                s = (1 + g) % 2
                s1 = 1 - s
                drain_out(i - 1, s1)
                fire_gather(i + 1, s1)
                drain_gather(i, s)
                fire_out(i, s)
            return carry

        lax.fori_loop(0, (n_groups - 2) // 2, steady, 0)

        last = n_groups - 1
        s_last = last % 2
        drain_gather(last, s_last)
        fire_out(last, s_last)
        drain_out(last - 1, 1 - s_last)
        drain_out(last, s_last)

    return k


def kernel(x, weight):
    batch, seq = x.shape
    n_total = batch * seq
    d = weight.shape[1]
    idx = x.reshape(_NW, n_total // (_NW * _G), _G).astype(jnp.int32)
    out = _make_gather_kernel(n_total, d)(idx, weight)
    return out.reshape(batch, seq, d)
